# Initial kernel scaffold; baseline (speedup 1.0000x reference)
import jax
import jax.numpy as jnp
from jax.experimental import pallas as pl


def kernel(x, edge_index, W1, b1, W2, b2, W3, b3, W4, b4, W5, b5, fc1_w, fc1_b, fc2_w, fc2_b):
    raise NotImplementedError("write your pallas kernel here")


# speedup vs baseline: 31.4471x; 31.4471x over previous
import functools

import jax
import jax.numpy as jnp
from jax import lax
from jax.experimental import pallas as pl
from jax.experimental.pallas import tpu as pltpu
from jax.experimental.pallas import tpu_sc as plsc

N = 81000
ACC_ROWS = 82000
E = 1620 * 1000
CH = 128
NTILE = 32
EPAD = ((E + NTILE * CH - 1) // (NTILE * CH)) * (NTILE * CH)
NCHUNK = EPAD // CH
CPT = NCHUNK // NTILE
ZR = 1000
R = 1000
GRID = N // R

_f32 = jnp.float32


def _sc_mesh():
    return plsc.VectorSubcoreMesh(core_axis_name="c", subcore_axis_name="s")


def _fill_rows(buf, nrows, value):
    vec = jnp.full((16,), value, _f32)

    def body(i, carry):
        buf[i, :] = vec
        return carry

    lax.fori_loop(0, nrows, body, 0)


def _zero_acc(acc, zbuf, s):
    nz = ACC_ROWS // ZR

    def body(t, carry):
        i = t * 16 + s

        @pl.when(i < nz)
        def _():
            pltpu.sync_copy(zbuf, acc.at[pl.ds(i * ZR, ZR)])

        return carry

    lax.fori_loop(0, (nz + 15) // 16, body, 0)


def _writeback(acc, out_slice, s):
    nw = N // ZR

    def body(t, carry):
        i = t * 16 + s

        @pl.when(i < nw)
        def _():
            pltpu.sync_copy(acc.at[pl.ds(i * ZR, ZR)],
                            out_slice.at[pl.ds(i * ZR, ZR)])

        return carry

    lax.fori_loop(0, (nw + 15) // 16, body, 0)


GP = 6
NG = CPT // GP


def _make_count():
    scratch = [
        pltpu.VMEM((GP, CH), jnp.int32),
        pltpu.VMEM((CH, 16), _f32),
        pltpu.VMEM((ZR, 16), _f32),
        pltpu.VMEM_SHARED((ACC_ROWS, 16), _f32),
        pltpu.SemaphoreType.DMA,
    ]

    @functools.partial(
        pl.kernel,
        out_type=jax.ShapeDtypeStruct((2, N, 16), _f32),
        mesh=_sc_mesh(),
        scratch_types=scratch,
        compiler_params=pltpu.CompilerParams(use_tc_tiling_on_sc=False),
    )
    def count(dst_hbm, out, dstb, ones_v, zbuf, acc, sem):
        c = lax.axis_index("c")
        s = lax.axis_index("s")
        w = s * 2 + c

        _fill_rows(ones_v, CH, 1.0)
        _fill_rows(zbuf, ZR, 0.0)
        _zero_acc(acc, zbuf, s)
        plsc.subcore_barrier()

        def body(g, carry):
            row0 = w * CPT + g * GP
            pltpu.sync_copy(dst_hbm.at[pl.ds(row0, GP)], dstb)
            scp = [pltpu.async_copy(ones_v, acc.at[dstb.at[b]], sem, add=True)
                   for b in range(GP)]
            for d in scp:
                d.wait()
            return carry

        lax.fori_loop(0, NG, body, 0)
        plsc.subcore_barrier()
        _writeback(acc, out.at[c], s)

    return count


def _make_prop(nb):
    scratch = [
        pltpu.VMEM((GP, CH), jnp.int32),
        pltpu.VMEM((GP, CH), jnp.int32),
        pltpu.VMEM((GP, CH, 16), _f32),
        pltpu.VMEM((ZR, 16), _f32),
        pltpu.VMEM_SHARED((ACC_ROWS, 16), _f32),
        pltpu.SemaphoreType.DMA,
        pltpu.SemaphoreType.DMA,
    ]

    @functools.partial(
        pl.kernel,
        out_type=jax.ShapeDtypeStruct((2, nb, N, 16), _f32),
        mesh=_sc_mesh(),
        scratch_types=scratch,
        compiler_params=pltpu.CompilerParams(use_tc_tiling_on_sc=False),
    )
    def prop(*refs):
        src_hbm, dst_hbm = refs[0], refs[1]
        gs = refs[2:2 + nb]
        out = refs[2 + nb]
        srcb, dstb, rows, zbuf, acc, semg, sems = refs[3 + nb:]

        c = lax.axis_index("c")
        s = lax.axis_index("s")
        w = s * 2 + c

        _fill_rows(zbuf, ZR, 0.0)

        for blk in range(nb):
            _zero_acc(acc, zbuf, s)
            plsc.subcore_barrier()

            def body(g, carry):
                row0 = w * CPT + g * GP
                pltpu.sync_copy(src_hbm.at[pl.ds(row0, GP)], srcb)
                pltpu.sync_copy(dst_hbm.at[pl.ds(row0, GP)], dstb)
                gcp = [pltpu.async_copy(gs[blk].at[srcb.at[b]], rows.at[b],
                                        semg)
                       for b in range(GP)]
                scp = []
                for b in range(GP):
                    gcp[b].wait()
                    scp.append(pltpu.async_copy(rows.at[b],
                                                acc.at[dstb.at[b]],
                                                sems, add=True))
                for d in scp:
                    d.wait()
                return carry

            lax.fori_loop(0, NG, body, 0)
            plsc.subcore_barrier()
            _writeback(acc, out.at[c, blk], s)
            plsc.subcore_barrier()

    return prop


_count = _make_count()
_prop1 = _make_prop(1)
_prop2 = _make_prop(2)


def _blk(rows, cols):
    return pl.BlockSpec((rows, cols), lambda i: (i, 0))


def _rep(shape):
    nd = len(shape)
    return pl.BlockSpec(shape, lambda i: (0,) * nd)


def _sblk(nb):
    return pl.BlockSpec((2, nb, R, 16), lambda i: (0, 0, i, 0))


def _out(cols):
    return jax.ShapeDtypeStruct((N, cols), _f32)


def _relu(v):
    return jnp.maximum(v, 0.0)


def _stage_a_body(cnt_ref, x_ref, dis_ref, g1_ref):
    cnt = cnt_ref[0, :, 0:1] + cnt_ref[1, :, 0:1]
    dis = lax.rsqrt(1.0 + cnt)
    xp = jnp.concatenate([x_ref[...], jnp.zeros((R, 6), _f32)], axis=1)
    dis_ref[...] = dis
    g1_ref[...] = dis * xp


_stage_a = pl.pallas_call(
    _stage_a_body,
    grid=(GRID,),
    in_specs=[pl.BlockSpec((2, R, 16), lambda i: (0, i, 0)), _blk(R, 10)],
    out_specs=[_blk(R, 1), _blk(R, 16)],
    out_shape=[_out(1), _out(16)],
)


def _stage_b1_body(s_ref, g_ref, dis_ref, w_ref, b_ref, out_ref):
    dis = dis_ref[...]
    u = dis * (s_ref[0, 0] + s_ref[1, 0] + g_ref[...])
    h = _relu(jnp.dot(u, w_ref[...], preferred_element_type=_f32) + b_ref[...])
    out_ref[...] = dis * h


_stage_b1 = pl.pallas_call(
    _stage_b1_body,
    grid=(GRID,),
    in_specs=[_sblk(1), _blk(R, 16), _blk(R, 1), _rep((16, 16)), _rep((1, 16))],
    out_specs=[_blk(R, 16)],
    out_shape=[_out(16)],
)


def _stage_b2_body(s_ref, g_ref, dis_ref, w_ref, b_ref, o0_ref, o1_ref):
    dis = dis_ref[...]
    u = dis * (s_ref[0, 0] + s_ref[1, 0] + g_ref[...])
    h = _relu(jnp.dot(u, w_ref[...], preferred_element_type=_f32) + b_ref[...])
    g = dis * h
    o0_ref[...] = g[:, :16]
    o1_ref[...] = g[:, 16:]


_stage_b2 = pl.pallas_call(
    _stage_b2_body,
    grid=(GRID,),
    in_specs=[_sblk(1), _blk(R, 16), _blk(R, 1), _rep((16, 32)), _rep((1, 32))],
    out_specs=[_blk(R, 16), _blk(R, 16)],
    out_shape=[_out(16), _out(16)],
)


def _stage_b3_body(s_ref, g0_ref, g1_ref, dis_ref, w3_ref, b3_ref, w4_ref,
                   o0_ref, o1_ref):
    dis = dis_ref[...]
    u0 = s_ref[0, 0] + s_ref[1, 0] + g0_ref[...]
    u1 = s_ref[0, 1] + s_ref[1, 1] + g1_ref[...]
    u = dis * jnp.concatenate([u0, u1], axis=1)
    h = _relu(jnp.dot(u, w3_ref[...], preferred_element_type=_f32)
              + b3_ref[...])
    g = dis * jnp.dot(h, w4_ref[...], preferred_element_type=_f32)
    o0_ref[...] = g[:, :16]
    o1_ref[...] = g[:, 16:]


_stage_b3 = pl.pallas_call(
    _stage_b3_body,
    grid=(GRID,),
    in_specs=[_sblk(2), _blk(R, 16), _blk(R, 16), _blk(R, 1),
              _rep((32, 64)), _rep((1, 64)), _rep((64, 32))],
    out_specs=[_blk(R, 16), _blk(R, 16)],
    out_shape=[_out(16), _out(16)],
)


def _stage_b4_body(s_ref, g0_ref, g1_ref, dis_ref, b4_ref, w5_ref, out_ref):
    dis = dis_ref[...]
    u0 = s_ref[0, 0] + s_ref[1, 0] + g0_ref[...]
    u1 = s_ref[0, 1] + s_ref[1, 1] + g1_ref[...]
    u = dis * jnp.concatenate([u0, u1], axis=1)
    h = _relu(u + b4_ref[...])
    out_ref[...] = dis * jnp.dot(h, w5_ref[...], preferred_element_type=_f32)


_stage_b4 = pl.pallas_call(
    _stage_b4_body,
    grid=(GRID,),
    in_specs=[_sblk(2), _blk(R, 16), _blk(R, 16), _blk(R, 1),
              _rep((1, 32)), _rep((32, 16))],
    out_specs=[_blk(R, 16)],
    out_shape=[_out(16)],
)


def _stage_b5_body(s_ref, g_ref, dis_ref, b5_ref, fc1w_ref, fc1b_ref,
                   fc2w_ref, fc2b_ref, out_ref):
    dis = dis_ref[...]
    h5 = _relu(dis * (s_ref[0, 0] + s_ref[1, 0] + g_ref[...]) + b5_ref[...])
    f = _relu(jnp.dot(h5, fc1w_ref[...], preferred_element_type=_f32)
              + fc1b_ref[...])
    out_ref[...] = (jnp.dot(f, fc2w_ref[...], preferred_element_type=_f32)
                    + fc2b_ref[...])


_stage_b5 = pl.pallas_call(
    _stage_b5_body,
    grid=(GRID,),
    in_specs=[_sblk(1), _blk(R, 16), _blk(R, 1), _rep((1, 16)),
              _rep((16, 16)), _rep((1, 16)), _rep((16, 9)), _rep((1, 9))],
    out_specs=[_blk(R, 9)],
    out_shape=[_out(9)],
)


def kernel(x, edge_index, W1, b1, W2, b2, W3, b3, W4, b4, W5, b5,
           fc1_w, fc1_b, fc2_w, fc2_b):
    pad = EPAD - E
    src = jnp.concatenate([edge_index[:, 0, :].reshape(-1),
                           jnp.zeros((pad,), jnp.int32)]).reshape(NCHUNK, CH)
    dst = jnp.concatenate([edge_index[:, 1, :].reshape(-1),
                           jnp.full((pad,), N, jnp.int32)]).reshape(NCHUNK, CH)

    cnt = _count(dst)
    dis, g1 = _stage_a(cnt, x)

    s1 = _prop1(src, dst, g1)
    g2 = _stage_b1(s1, g1, dis, jnp.pad(W1, ((0, 6), (0, 0))),
                   b1.reshape(1, -1))[0]

    s2 = _prop1(src, dst, g2)
    g3_0, g3_1 = _stage_b2(s2, g2, dis, W2, b2.reshape(1, -1))

    s3 = _prop2(src, dst, g3_0, g3_1)
    g4_0, g4_1 = _stage_b3(s3, g3_0, g3_1, dis, W3, b3.reshape(1, -1), W4)

    s4 = _prop2(src, dst, g4_0, g4_1)
    g5 = _stage_b4(s4, g4_0, g4_1, dis, b4.reshape(1, -1), W5)[0]

    s5 = _prop1(src, dst, g5)
    y = _stage_b5(s5, g5, dis, b5.reshape(1, -1), fc1_w,
                  fc1_b.reshape(1, -1), fc2_w, fc2_b.reshape(1, -1))[0]

    return y.reshape(-1, 81, 9)
